# baseline (speedup 1.0000x reference)
import functools
import math

import numpy as np
import jax
import jax.numpy as jnp
from jax import lax
from jax.experimental import pallas as pl
from jax.experimental.pallas import tpu as pltpu
from jax.experimental.pallas import tpu_sc as plsc

VOCAB = 1000000
D = 64
B = 4096
L = 200

NC = 2
NS = 16
NW = NC * NS
SEQ_PER_W = B // NW

SCALE = 2.0 * math.sqrt(D)


def _make_pe() -> jnp.ndarray:
    position = np.arange(0, L, dtype=np.float32)[:, None]
    div_even = np.power(10000.0, np.arange(0, D, 2, dtype=np.float32) / D)
    div_odd = np.power(10000.0, np.arange(1, D, 2, dtype=np.float32) / D)
    pe = np.zeros((L, D), dtype=np.float32)
    pe[:, 0::2] = np.sin(position * div_even)
    pe[:, 1::2] = np.cos(position * div_odd)
    return jnp.asarray(pe)


_PE = _make_pe()


@functools.cache
def _build():
    mesh = plsc.VectorSubcoreMesh(
        core_axis_name="c", subcore_axis_name="s", num_cores=NC, num_subcores=NS
    )

    @functools.partial(
        pl.kernel,
        mesh=mesh,
        out_type=jax.ShapeDtypeStruct((B, L, D), jnp.float32),
        scratch_types=[
            pltpu.VMEM((L,), jnp.int32),
            pltpu.VMEM((L, D), jnp.float32),
            pltpu.VMEM((L, D), jnp.float32),
            pltpu.SemaphoreType.DMA,
        ],
        compiler_params=pltpu.CompilerParams(use_tc_tiling_on_sc=False),
    )
    def emb(x_hbm, table_hbm, pe_hbm, out_hbm, idx_v, rows_v, pe_v, sem):
        wid = lax.axis_index("s") * NC + lax.axis_index("c")
        base = wid * SEQ_PER_W
        pltpu.sync_copy(pe_hbm, pe_v)

        def per_seq(i, carry):
            seq = base + i
            pltpu.sync_copy(x_hbm.at[seq], idx_v)
            pltpu.async_copy(table_hbm.at[idx_v], rows_v, sem).wait()

            def per_row(l, c):
                for k in range(D // 16):
                    sl = pl.ds(k * 16, 16)
                    rows_v[l, sl] = rows_v[l, sl] * SCALE + pe_v[l, sl]
                return c

            lax.fori_loop(0, L, per_row, 0)
            pltpu.sync_copy(rows_v, out_hbm.at[seq])
            return carry

        lax.fori_loop(0, SEQ_PER_W, per_seq, 0)

    return emb


def kernel(x, table):
    return _build()(x, table, _PE)


# speedup vs baseline: 1.1800x; 1.1800x over previous
import functools
import math

import numpy as np
import jax
import jax.numpy as jnp
from jax import lax
from jax.experimental import pallas as pl
from jax.experimental.pallas import tpu as pltpu
from jax.experimental.pallas import tpu_sc as plsc

VOCAB = 1000000
D = 64
B = 4096
L = 200

NC = 2
NS = 16
NW = NC * NS
SEQ_PER_W = B // NW
NBUF = 4
LOOKAHEAD = 2

SCALE = 2.0 * math.sqrt(D)


def _make_pe() -> jnp.ndarray:
    position = np.arange(0, L, dtype=np.float32)[:, None]
    div_even = np.power(10000.0, np.arange(0, D, 2, dtype=np.float32) / D)
    div_odd = np.power(10000.0, np.arange(1, D, 2, dtype=np.float32) / D)
    pe = np.zeros((L, D), dtype=np.float32)
    pe[:, 0::2] = np.sin(position * div_even)
    pe[:, 1::2] = np.cos(position * div_odd)
    return jnp.asarray(pe)


_PE = _make_pe()


@functools.cache
def _build():
    mesh = plsc.VectorSubcoreMesh(
        core_axis_name="c", subcore_axis_name="s", num_cores=NC, num_subcores=NS
    )

    @functools.partial(
        pl.kernel,
        mesh=mesh,
        out_type=jax.ShapeDtypeStruct((B, L, D), jnp.float32),
        scratch_types=[
            [pltpu.VMEM((L,), jnp.int32) for _ in range(NBUF)],
            [pltpu.VMEM((L, D), jnp.float32) for _ in range(NBUF)],
            pltpu.VMEM((L, D), jnp.float32),
            [pltpu.SemaphoreType.DMA for _ in range(NBUF)],
            [pltpu.SemaphoreType.DMA for _ in range(NBUF)],
        ],
        compiler_params=pltpu.CompilerParams(use_tc_tiling_on_sc=False),
    )
    def emb(x_hbm, table_hbm, pe_hbm, out_hbm, idx_v, rows_v, pe_v, gsem, ssem):
        wid = lax.axis_index("s") * NC + lax.axis_index("c")
        base = wid * SEQ_PER_W
        pltpu.sync_copy(pe_hbm, pe_v)

        def start_gather(b, seq):
            pltpu.sync_copy(x_hbm.at[seq], idx_v[b])
            pltpu.async_copy(table_hbm.at[idx_v[b]], rows_v[b], gsem[b])

        for j in range(LOOKAHEAD):
            start_gather(j, base + j)

        def body(i, carry):
            for j in range(NBUF):
                s = i * NBUF + j
                b = j
                ba = (j + LOOKAHEAD) % NBUF

                @pl.when(s + LOOKAHEAD < SEQ_PER_W)
                def _():
                    @pl.when(s >= NBUF - LOOKAHEAD)
                    def _():
                        pltpu.make_async_copy(
                            rows_v[ba], out_hbm.at[base + s], ssem[ba]
                        ).wait()

                    start_gather(ba, base + s + LOOKAHEAD)

                pltpu.make_async_copy(
                    table_hbm.at[idx_v[b]], rows_v[b], gsem[b]
                ).wait()

                def per_rows(r4, c):
                    for r in range(4):
                        row = r4 * 4 + r
                        for k in range(D // 16):
                            sl = pl.ds(k * 16, 16)
                            rows_v[b][row, sl] = (
                                rows_v[b][row, sl] * SCALE + pe_v[row, sl]
                            )
                    return c

                lax.fori_loop(0, L // 4, per_rows, 0)
                pltpu.async_copy(rows_v[b], out_hbm.at[base + s], ssem[b])
            return carry

        lax.fori_loop(0, SEQ_PER_W // NBUF, body, 0)

        for j in range(NBUF):
            pltpu.make_async_copy(
                rows_v[j], out_hbm.at[base + SEQ_PER_W - NBUF + j], ssem[j]
            ).wait()

    return emb


def kernel(x, table):
    return _build()(x, table, _PE)
